# baseline (speedup 1.0000x reference)
import jax
import jax.numpy as jnp
from jax import lax
from jax.experimental import pallas as pl
from jax.experimental.pallas import tpu as pltpu
from jax.experimental.pallas import tpu_sc as plsc

B = 16384
D = 128
C = 64
O = 10
K = 8
TB = 256
NC = 2
NS = 16
NW = NC * NS
RW = B // NW
GG = RW // 16


def _tc1_kernel(x_ref, keys_ref, wv_ref, bv_ref, wt_ref, bt_ref,
                cos_out, cost_out, cosd_out, van_out, tanh_out):
    x = x_ref[...]
    norm = jnp.sqrt(jnp.sum(x * x, axis=1, keepdims=True))
    xn = x / jnp.maximum(norm, 1e-12)

    dn = (((1,), (1,)), ((), ()))
    cos = lax.dot_general(xn, keys_ref[...], dn,
                          preferred_element_type=jnp.float32)
    cos_out[...] = cos
    cost_out[...] = cos.T
    cosd_out[...] = 1.0 - cos

    lv = lax.dot_general(x, wv_ref[...], dn,
                         preferred_element_type=jnp.float32) + bv_ref[...]
    m = jnp.max(lv, axis=1, keepdims=True)
    sh = lv - m
    van_out[...] = sh - jnp.log(jnp.sum(jnp.exp(sh), axis=1, keepdims=True))

    lt = lax.dot_general(x, wt_ref[...], dn,
                         preferred_element_type=jnp.float32) + bt_ref[...]
    tanh_out[...] = jnp.tanh(lt)


def _sc_topk_body(cost_hbm, maskt_hbm, in_v, out_v):
    wid = lax.axis_index("s") * NC + lax.axis_index("c")
    base = wid * RW
    pltpu.sync_copy(cost_hbm.at[:, pl.ds(base, RW)], in_v)

    neg_inf = jnp.full((16,), -jnp.inf, jnp.float32)
    onesf = jnp.full((16,), 1.0, jnp.float32)
    zerosf = jnp.zeros((16,), jnp.float32)
    zerosi = jnp.zeros((16,), jnp.int32)

    @plsc.parallel_loop(0, GG, step=1)
    def group_body(g):
        sl = pl.ds(g * 16, 16)
        hist = []
        prev = None
        for _ in range(K):
            m = neg_inf
            midx = zerosi
            for c in range(C):
                v = in_v[c, sl]
                if prev is not None:
                    v = jnp.where(prev == c, -jnp.inf, v)
                    in_v[c, sl] = v
                gt = v > m
                m = jnp.where(gt, v, m)
                midx = jnp.where(gt, c, midx)
            hist.append(midx)
            prev = midx
        for c in range(C):
            hit = hist[0] == c
            for h in hist[1:]:
                hit = hit | (h == c)
            out_v[c, sl] = jnp.where(hit, onesf, zerosf)

    pltpu.sync_copy(out_v, maskt_hbm.at[:, pl.ds(base, RW)])


def _tc2_kernel(x_ref, cos_ref, maskt_ref, wm_ref, bm_ref, sel_ref,
                ens_out, knn_out):
    x = x_ref[...]
    cos = cos_ref[...]
    maskf = maskt_ref[...].T
    knn_out[...] = maskf
    w = cos * maskf
    denom = jnp.sum(w, axis=1, keepdims=True)

    dn = (((1,), (1,)), ((), ()))
    z = lax.dot_general(x, wm_ref[...], dn,
                        preferred_element_type=jnp.float32) + bm_ref[...]
    ens = jnp.tanh(z)
    w_tiled = jnp.concatenate([w] * O, axis=1)
    p = ens * w_tiled
    num = lax.dot_general(
        p, sel_ref[...], (((1,), (0,)), ((), ())),
        preferred_element_type=jnp.float32)
    ens_out[...] = num / denom


@jax.jit
def kernel(x, keys, W_models, b_models, W_van, b_van, W_tanh, b_tanh):
    wm_perm = W_models.transpose(1, 0, 2).reshape(O * C, D)
    bm_perm = b_models.T.reshape(1, O * C)
    sel = (jnp.arange(O * C)[:, None] // C ==
           jnp.arange(O)[None, :]).astype(jnp.float32)
    bv = b_van.reshape(1, O)
    bt = b_tanh.reshape(1, O)

    grid = (B // TB,)
    f32 = jnp.float32

    cos, cost, cosd, van, tanh_o = pl.pallas_call(
        _tc1_kernel,
        grid=grid,
        in_specs=[
            pl.BlockSpec((TB, D), lambda i: (i, 0)),
            pl.BlockSpec((C, D), lambda i: (0, 0)),
            pl.BlockSpec((O, D), lambda i: (0, 0)),
            pl.BlockSpec((1, O), lambda i: (0, 0)),
            pl.BlockSpec((O, D), lambda i: (0, 0)),
            pl.BlockSpec((1, O), lambda i: (0, 0)),
        ],
        out_specs=[
            pl.BlockSpec((TB, C), lambda i: (i, 0)),
            pl.BlockSpec((C, TB), lambda i: (0, i)),
            pl.BlockSpec((TB, C), lambda i: (i, 0)),
            pl.BlockSpec((TB, O), lambda i: (i, 0)),
            pl.BlockSpec((TB, O), lambda i: (i, 0)),
        ],
        out_shape=[
            jax.ShapeDtypeStruct((B, C), f32),
            jax.ShapeDtypeStruct((C, B), f32),
            jax.ShapeDtypeStruct((B, C), f32),
            jax.ShapeDtypeStruct((B, O), f32),
            jax.ShapeDtypeStruct((B, O), f32),
        ],
    )(x, keys, W_van, bv, W_tanh, bt)

    mesh = plsc.VectorSubcoreMesh(core_axis_name="c", subcore_axis_name="s",
                                  num_cores=NC, num_subcores=NS)
    maskt = pl.kernel(
        _sc_topk_body,
        out_type=jax.ShapeDtypeStruct((C, B), f32),
        mesh=mesh,
        scratch_types=[
            pltpu.VMEM((C, RW), f32),
            pltpu.VMEM((C, RW), f32),
        ],
    )(cost)

    ens_o, knn = pl.pallas_call(
        _tc2_kernel,
        grid=grid,
        in_specs=[
            pl.BlockSpec((TB, D), lambda i: (i, 0)),
            pl.BlockSpec((TB, C), lambda i: (i, 0)),
            pl.BlockSpec((C, TB), lambda i: (0, i)),
            pl.BlockSpec((O * C, D), lambda i: (0, 0)),
            pl.BlockSpec((1, O * C), lambda i: (0, 0)),
            pl.BlockSpec((O * C, O), lambda i: (0, 0)),
        ],
        out_specs=[
            pl.BlockSpec((TB, O), lambda i: (i, 0)),
            pl.BlockSpec((TB, C), lambda i: (i, 0)),
        ],
        out_shape=[
            jax.ShapeDtypeStruct((B, O), f32),
            jax.ShapeDtypeStruct((B, C), f32),
        ],
    )(x, cos, maskt, wm_perm, bm_perm, sel)

    return (ens_o, tanh_o, van, cosd, knn)


# speedup vs baseline: 1.1124x; 1.1124x over previous
import jax
import jax.numpy as jnp
from jax import lax
from jax.experimental import pallas as pl
from jax.experimental.pallas import tpu as pltpu
from jax.experimental.pallas import tpu_sc as plsc

B = 16384
D = 128
C = 64
O = 10
K = 8
TB = 256
NC = 2
NS = 16
NW = NC * NS
RW = B // NW
GG = RW // 16


def _tc1_kernel(x_ref, keys_ref, wv_ref, bv_ref, wt_ref, bt_ref,
                cos_out, cost_out, cosd_out, van_out, tanh_out):
    x = x_ref[...]
    norm = jnp.sqrt(jnp.sum(x * x, axis=1, keepdims=True))
    xn = x / jnp.maximum(norm, 1e-12)

    dn = (((1,), (1,)), ((), ()))
    cos = lax.dot_general(xn, keys_ref[...], dn,
                          preferred_element_type=jnp.float32)
    cos_out[...] = cos
    cost_out[...] = cos.T
    cosd_out[...] = 1.0 - cos

    lv = lax.dot_general(x, wv_ref[...], dn,
                         preferred_element_type=jnp.float32) + bv_ref[...]
    m = jnp.max(lv, axis=1, keepdims=True)
    sh = lv - m
    van_out[...] = sh - jnp.log(jnp.sum(jnp.exp(sh), axis=1, keepdims=True))

    lt = lax.dot_general(x, wt_ref[...], dn,
                         preferred_element_type=jnp.float32) + bt_ref[...]
    tanh_out[...] = jnp.tanh(lt)


def _sc_topk_body(cost_hbm, maskt_hbm, in_v, out_v):
    wid = lax.axis_index("s") * NC + lax.axis_index("c")
    base = wid * RW
    pltpu.sync_copy(cost_hbm.at[:, pl.ds(base, RW)], in_v)

    neg_inf = jnp.full((16,), -jnp.inf, jnp.float32)
    onesf = jnp.full((16,), 1.0, jnp.float32)
    zerosf = jnp.zeros((16,), jnp.float32)
    zerosi = jnp.zeros((16,), jnp.int32)

    @plsc.parallel_loop(0, GG, step=1)
    def group_body(g):
        sl = pl.ds(g * 16, 16)
        hist = []
        prev = None
        for _ in range(K):
            ms = [neg_inf] * 4
            idxs = [zerosi] * 4
            for cc in range(16):
                for s in range(4):
                    c = s * 16 + cc
                    v = in_v[c, sl]
                    if prev is not None:
                        v = jnp.where(prev == c, -jnp.inf, v)
                        in_v[c, sl] = v
                    gt = v > ms[s]
                    ms[s] = jnp.where(gt, v, ms[s])
                    idxs[s] = jnp.where(gt, c, idxs[s])
            a_ge = ms[0] >= ms[1]
            m01 = jnp.where(a_ge, ms[0], ms[1])
            i01 = jnp.where(a_ge, idxs[0], idxs[1])
            b_ge = ms[2] >= ms[3]
            m23 = jnp.where(b_ge, ms[2], ms[3])
            i23 = jnp.where(b_ge, idxs[2], idxs[3])
            f_ge = m01 >= m23
            midx = jnp.where(f_ge, i01, i23)
            hist.append(midx)
            prev = midx
        for c in range(C):
            hit = hist[0] == c
            for h in hist[1:]:
                hit = hit | (h == c)
            out_v[c, sl] = jnp.where(hit, onesf, zerosf)

    pltpu.sync_copy(out_v, maskt_hbm.at[:, pl.ds(base, RW)])


def _tc2_kernel(x_ref, cos_ref, maskt_ref, wm_ref, bm_ref, sel_ref,
                ens_out, knn_out):
    x = x_ref[...]
    cos = cos_ref[...]
    maskf = maskt_ref[...].T
    knn_out[...] = maskf
    w = cos * maskf
    denom = jnp.sum(w, axis=1, keepdims=True)

    dn = (((1,), (1,)), ((), ()))
    z = lax.dot_general(x, wm_ref[...], dn,
                        preferred_element_type=jnp.float32) + bm_ref[...]
    ens = jnp.tanh(z)
    w_tiled = jnp.concatenate([w] * O, axis=1)
    p = ens * w_tiled
    num = lax.dot_general(
        p, sel_ref[...], (((1,), (0,)), ((), ())),
        preferred_element_type=jnp.float32)
    ens_out[...] = num / denom


@jax.jit
def kernel(x, keys, W_models, b_models, W_van, b_van, W_tanh, b_tanh):
    wm_perm = W_models.transpose(1, 0, 2).reshape(O * C, D)
    bm_perm = b_models.T.reshape(1, O * C)
    sel = (jnp.arange(O * C)[:, None] // C ==
           jnp.arange(O)[None, :]).astype(jnp.float32)
    bv = b_van.reshape(1, O)
    bt = b_tanh.reshape(1, O)

    grid = (B // TB,)
    f32 = jnp.float32

    cos, cost, cosd, van, tanh_o = pl.pallas_call(
        _tc1_kernel,
        grid=grid,
        in_specs=[
            pl.BlockSpec((TB, D), lambda i: (i, 0)),
            pl.BlockSpec((C, D), lambda i: (0, 0)),
            pl.BlockSpec((O, D), lambda i: (0, 0)),
            pl.BlockSpec((1, O), lambda i: (0, 0)),
            pl.BlockSpec((O, D), lambda i: (0, 0)),
            pl.BlockSpec((1, O), lambda i: (0, 0)),
        ],
        out_specs=[
            pl.BlockSpec((TB, C), lambda i: (i, 0)),
            pl.BlockSpec((C, TB), lambda i: (0, i)),
            pl.BlockSpec((TB, C), lambda i: (i, 0)),
            pl.BlockSpec((TB, O), lambda i: (i, 0)),
            pl.BlockSpec((TB, O), lambda i: (i, 0)),
        ],
        out_shape=[
            jax.ShapeDtypeStruct((B, C), f32),
            jax.ShapeDtypeStruct((C, B), f32),
            jax.ShapeDtypeStruct((B, C), f32),
            jax.ShapeDtypeStruct((B, O), f32),
            jax.ShapeDtypeStruct((B, O), f32),
        ],
    )(x, keys, W_van, bv, W_tanh, bt)

    mesh = plsc.VectorSubcoreMesh(core_axis_name="c", subcore_axis_name="s",
                                  num_cores=NC, num_subcores=NS)
    maskt = pl.kernel(
        _sc_topk_body,
        out_type=jax.ShapeDtypeStruct((C, B), f32),
        mesh=mesh,
        scratch_types=[
            pltpu.VMEM((C, RW), f32),
            pltpu.VMEM((C, RW), f32),
        ],
    )(cost)

    ens_o, knn = pl.pallas_call(
        _tc2_kernel,
        grid=grid,
        in_specs=[
            pl.BlockSpec((TB, D), lambda i: (i, 0)),
            pl.BlockSpec((TB, C), lambda i: (i, 0)),
            pl.BlockSpec((C, TB), lambda i: (0, i)),
            pl.BlockSpec((O * C, D), lambda i: (0, 0)),
            pl.BlockSpec((1, O * C), lambda i: (0, 0)),
            pl.BlockSpec((O * C, O), lambda i: (0, 0)),
        ],
        out_specs=[
            pl.BlockSpec((TB, O), lambda i: (i, 0)),
            pl.BlockSpec((TB, C), lambda i: (i, 0)),
        ],
        out_shape=[
            jax.ShapeDtypeStruct((B, O), f32),
            jax.ShapeDtypeStruct((B, C), f32),
        ],
    )(x, cos, maskt, wm_perm, bm_perm, sel)

    return (ens_o, tanh_o, van, cosd, knn)


# speedup vs baseline: 1.5538x; 1.3968x over previous
import jax
import jax.numpy as jnp
from jax import lax
from jax.experimental import pallas as pl
from jax.experimental.pallas import tpu as pltpu
from jax.experimental.pallas import tpu_sc as plsc

B = 16384
D = 128
C = 64
O = 10
K = 8
TB = 512
NC = 2
NS = 16
NW = NC * NS
RW = B // NW


def _tc1_kernel(x_ref, keys_ref, wv_ref, bv_ref, wt_ref, bt_ref,
                cos_out, cosd_out, van_out, tanh_out):
    x = x_ref[...]
    norm = jnp.sqrt(jnp.sum(x * x, axis=1, keepdims=True))
    xn = x / jnp.maximum(norm, 1e-12)

    dn = (((1,), (1,)), ((), ()))
    cos = lax.dot_general(xn, keys_ref[...], dn,
                          preferred_element_type=jnp.float32)
    cos_out[...] = cos
    cosd_out[...] = 1.0 - cos

    lv = lax.dot_general(x, wv_ref[...], dn,
                         preferred_element_type=jnp.float32) + bv_ref[...]
    m = jnp.max(lv, axis=1, keepdims=True)
    sh = lv - m
    van_out[...] = sh - jnp.log(jnp.sum(jnp.exp(sh), axis=1, keepdims=True))

    lt = lax.dot_general(x, wt_ref[...], dn,
                         preferred_element_type=jnp.float32) + bt_ref[...]
    tanh_out[...] = jnp.tanh(lt)


_PERM_DNUMS = lax.GatherDimensionNumbers(
    offset_dims=(), collapsed_slice_dims=(0,), start_index_map=(0,))


def _lane_perm(v, perm_idx):
    return lax.gather(v, perm_idx[:, None], dimension_numbers=_PERM_DNUMS,
                      slice_sizes=(1,),
                      mode=lax.GatherScatterMode.PROMISE_IN_BOUNDS)


def _sc_topk_body(cos_hbm, mask_hbm, in_v, out_v):
    wid = lax.axis_index("s") * NC + lax.axis_index("c")
    base = wid * (RW * C)
    pltpu.sync_copy(cos_hbm.at[pl.ds(base, RW * C)], in_v)

    iota = lax.broadcasted_iota(jnp.int32, (16,), 0)
    perms = [iota ^ p for p in (8, 4, 2, 1)]
    lane_ids = [iota + 16 * j for j in range(4)]
    neg_inf = jnp.full((16,), -jnp.inf, jnp.float32)
    big = jnp.full((16,), 1000, jnp.int32)
    ones = jnp.full((16,), 1.0, jnp.float32)
    zeros = jnp.zeros((16,), jnp.float32)

    @plsc.parallel_loop(0, RW, step=1, unroll=8)
    def row_body(r):
        off = r * C
        vs = [in_v[pl.ds(off + 16 * j, 16)] for j in range(4)]
        ms = [zeros, zeros, zeros, zeros]
        for _ in range(K):
            m = jnp.maximum(jnp.maximum(vs[0], vs[1]),
                            jnp.maximum(vs[2], vs[3]))
            for p in perms:
                m = jnp.maximum(m, _lane_perm(m, p))
            cands = [jnp.where(vs[j] == m, lane_ids[j], big)
                     for j in range(4)]
            g = jnp.minimum(jnp.minimum(cands[0], cands[1]),
                            jnp.minimum(cands[2], cands[3]))
            for p in perms:
                g = jnp.minimum(g, _lane_perm(g, p))
            for j in range(4):
                s = lane_ids[j] == g
                vs[j] = jnp.where(s, neg_inf, vs[j])
                ms[j] = jnp.where(s, ones, ms[j])
        for j in range(4):
            out_v[pl.ds(off + 16 * j, 16)] = ms[j]

    pltpu.sync_copy(out_v, mask_hbm.at[pl.ds(base, RW * C)])


def _tc2_kernel(x_ref, cos_ref, mask_ref, wm_ref, bm_ref, sel_ref, ens_out):
    x = x_ref[...]
    cos = cos_ref[...]
    maskf = mask_ref[...]
    w = cos * maskf
    denom = jnp.sum(w, axis=1, keepdims=True)

    dn = (((1,), (1,)), ((), ()))
    z = lax.dot_general(x, wm_ref[...], dn,
                        preferred_element_type=jnp.float32) + bm_ref[...]
    ens = jnp.tanh(z)
    w_tiled = jnp.concatenate([w] * O, axis=1)
    p = ens * w_tiled
    num = lax.dot_general(
        p, sel_ref[...], (((1,), (0,)), ((), ())),
        preferred_element_type=jnp.float32)
    ens_out[...] = num / denom


@jax.jit
def kernel(x, keys, W_models, b_models, W_van, b_van, W_tanh, b_tanh):
    wm_perm = W_models.transpose(1, 0, 2).reshape(O * C, D)
    bm_perm = b_models.T.reshape(1, O * C)
    sel = (jnp.arange(O * C)[:, None] // C ==
           jnp.arange(O)[None, :]).astype(jnp.float32)
    bv = b_van.reshape(1, O)
    bt = b_tanh.reshape(1, O)

    grid = (B // TB,)
    f32 = jnp.float32

    cos, cosd, van, tanh_o = pl.pallas_call(
        _tc1_kernel,
        grid=grid,
        in_specs=[
            pl.BlockSpec((TB, D), lambda i: (i, 0)),
            pl.BlockSpec((C, D), lambda i: (0, 0)),
            pl.BlockSpec((O, D), lambda i: (0, 0)),
            pl.BlockSpec((1, O), lambda i: (0, 0)),
            pl.BlockSpec((O, D), lambda i: (0, 0)),
            pl.BlockSpec((1, O), lambda i: (0, 0)),
        ],
        out_specs=[
            pl.BlockSpec((TB, C), lambda i: (i, 0)),
            pl.BlockSpec((TB, C), lambda i: (i, 0)),
            pl.BlockSpec((TB, O), lambda i: (i, 0)),
            pl.BlockSpec((TB, O), lambda i: (i, 0)),
        ],
        out_shape=[
            jax.ShapeDtypeStruct((B, C), f32),
            jax.ShapeDtypeStruct((B, C), f32),
            jax.ShapeDtypeStruct((B, O), f32),
            jax.ShapeDtypeStruct((B, O), f32),
        ],
    )(x, keys, W_van, bv, W_tanh, bt)

    mesh = plsc.VectorSubcoreMesh(core_axis_name="c", subcore_axis_name="s",
                                  num_cores=NC, num_subcores=NS)
    mask_flat = pl.kernel(
        _sc_topk_body,
        out_type=jax.ShapeDtypeStruct((B * C,), f32),
        mesh=mesh,
        scratch_types=[
            pltpu.VMEM((RW * C,), f32),
            pltpu.VMEM((RW * C,), f32),
        ],
    )(cos.reshape(B * C))
    knn = mask_flat.reshape(B, C)

    ens_o = pl.pallas_call(
        _tc2_kernel,
        grid=grid,
        in_specs=[
            pl.BlockSpec((TB, D), lambda i: (i, 0)),
            pl.BlockSpec((TB, C), lambda i: (i, 0)),
            pl.BlockSpec((TB, C), lambda i: (i, 0)),
            pl.BlockSpec((O * C, D), lambda i: (0, 0)),
            pl.BlockSpec((1, O * C), lambda i: (0, 0)),
            pl.BlockSpec((O * C, O), lambda i: (0, 0)),
        ],
        out_specs=pl.BlockSpec((TB, O), lambda i: (i, 0)),
        out_shape=jax.ShapeDtypeStruct((B, O), f32),
    )(x, cos, knn, wm_perm, bm_perm, sel)

    return (ens_o, tanh_o, van, cosd, knn)


# speedup vs baseline: 1.7675x; 1.1375x over previous
import jax
import jax.numpy as jnp
from jax import lax
from jax.experimental import pallas as pl
from jax.experimental.pallas import tpu as pltpu
from jax.experimental.pallas import tpu_sc as plsc

B = 16384
D = 128
C = 64
O = 10
K = 8
TB = 1024
NC = 2
NS = 16
NW = NC * NS
RW = B // NW


def _tc1_kernel(x_ref, keys_ref, wv_ref, bv_ref, wt_ref, bt_ref,
                cos_out, cosd_out, van_out, tanh_out):
    x = x_ref[...]
    norm = jnp.sqrt(jnp.sum(x * x, axis=1, keepdims=True))
    xn = x / jnp.maximum(norm, 1e-12)

    dn = (((1,), (1,)), ((), ()))
    cos = lax.dot_general(xn, keys_ref[...], dn,
                          preferred_element_type=jnp.float32)
    cos_out[...] = cos
    cosd_out[...] = 1.0 - cos

    lv = lax.dot_general(x, wv_ref[...], dn,
                         preferred_element_type=jnp.float32) + bv_ref[...]
    m = jnp.max(lv, axis=1, keepdims=True)
    sh = lv - m
    van_out[...] = sh - jnp.log(jnp.sum(jnp.exp(sh), axis=1, keepdims=True))

    lt = lax.dot_general(x, wt_ref[...], dn,
                         preferred_element_type=jnp.float32) + bt_ref[...]
    tanh_out[...] = jnp.tanh(lt)


_PERM_DNUMS = lax.GatherDimensionNumbers(
    offset_dims=(), collapsed_slice_dims=(0,), start_index_map=(0,))


def _lane_perm(v, perm_idx):
    return lax.gather(v, perm_idx[:, None], dimension_numbers=_PERM_DNUMS,
                      slice_sizes=(1,),
                      mode=lax.GatherScatterMode.PROMISE_IN_BOUNDS)


def _sc_topk_body(cos_hbm, mask_hbm, in_v, out_v):
    wid = lax.axis_index("s") * NC + lax.axis_index("c")
    base = wid * (RW * C)
    pltpu.sync_copy(cos_hbm.at[pl.ds(base, RW * C)], in_v)

    iota = lax.broadcasted_iota(jnp.int32, (16,), 0)
    perms = [iota ^ p for p in (8, 4, 2, 1)]
    lane_ids = [iota + 16 * j for j in range(4)]
    neg_inf = jnp.full((16,), -jnp.inf, jnp.float32)
    big = jnp.full((16,), 1000, jnp.int32)
    ones = jnp.full((16,), 1.0, jnp.float32)
    zeros = jnp.zeros((16,), jnp.float32)

    @plsc.parallel_loop(0, RW, step=1, unroll=8)
    def row_body(r):
        off = r * C
        vs = [in_v[pl.ds(off + 16 * j, 16)] for j in range(4)]
        ms = [zeros, zeros, zeros, zeros]
        for _ in range(K):
            m = jnp.maximum(jnp.maximum(vs[0], vs[1]),
                            jnp.maximum(vs[2], vs[3]))
            for p in perms:
                m = jnp.maximum(m, _lane_perm(m, p))
            cands = [jnp.where(vs[j] == m, lane_ids[j], big)
                     for j in range(4)]
            g = jnp.minimum(jnp.minimum(cands[0], cands[1]),
                            jnp.minimum(cands[2], cands[3]))
            for p in perms:
                g = jnp.minimum(g, _lane_perm(g, p))
            for j in range(4):
                s = lane_ids[j] == g
                vs[j] = jnp.where(s, neg_inf, vs[j])
                ms[j] = jnp.where(s, ones, ms[j])
        for j in range(4):
            out_v[pl.ds(off + 16 * j, 16)] = ms[j]

    pltpu.sync_copy(out_v, mask_hbm.at[pl.ds(base, RW * C)])


def _tc2_kernel(x_ref, cos_ref, mask_ref, wm_ref, bm_ref, sel_ref, ens_out):
    x = x_ref[...]
    cos = cos_ref[...]
    maskf = mask_ref[...]
    w = cos * maskf
    denom = jnp.sum(w, axis=1, keepdims=True)

    dn = (((1,), (1,)), ((), ()))
    z = lax.dot_general(x, wm_ref[...], dn,
                        preferred_element_type=jnp.float32) + bm_ref[...]
    ens = jnp.tanh(z)
    w_tiled = jnp.concatenate([w] * O, axis=1)
    p = ens * w_tiled
    num = lax.dot_general(
        p, sel_ref[...], (((1,), (0,)), ((), ())),
        preferred_element_type=jnp.float32)
    ens_out[...] = num / denom


@jax.jit
def kernel(x, keys, W_models, b_models, W_van, b_van, W_tanh, b_tanh):
    wm_perm = W_models.transpose(1, 0, 2).reshape(O * C, D)
    bm_perm = b_models.T.reshape(1, O * C)
    sel = (jnp.arange(O * C)[:, None] // C ==
           jnp.arange(O)[None, :]).astype(jnp.float32)
    bv = b_van.reshape(1, O)
    bt = b_tanh.reshape(1, O)

    grid = (B // TB,)
    f32 = jnp.float32

    cos, cosd, van, tanh_o = pl.pallas_call(
        _tc1_kernel,
        grid=grid,
        in_specs=[
            pl.BlockSpec((TB, D), lambda i: (i, 0)),
            pl.BlockSpec((C, D), lambda i: (0, 0)),
            pl.BlockSpec((O, D), lambda i: (0, 0)),
            pl.BlockSpec((1, O), lambda i: (0, 0)),
            pl.BlockSpec((O, D), lambda i: (0, 0)),
            pl.BlockSpec((1, O), lambda i: (0, 0)),
        ],
        out_specs=[
            pl.BlockSpec((TB, C), lambda i: (i, 0)),
            pl.BlockSpec((TB, C), lambda i: (i, 0)),
            pl.BlockSpec((TB, O), lambda i: (i, 0)),
            pl.BlockSpec((TB, O), lambda i: (i, 0)),
        ],
        out_shape=[
            jax.ShapeDtypeStruct((B, C), f32),
            jax.ShapeDtypeStruct((B, C), f32),
            jax.ShapeDtypeStruct((B, O), f32),
            jax.ShapeDtypeStruct((B, O), f32),
        ],
    )(x, keys, W_van, bv, W_tanh, bt)

    mesh = plsc.VectorSubcoreMesh(core_axis_name="c", subcore_axis_name="s",
                                  num_cores=NC, num_subcores=NS)
    mask_flat = pl.kernel(
        _sc_topk_body,
        out_type=jax.ShapeDtypeStruct((B * C,), f32),
        mesh=mesh,
        scratch_types=[
            pltpu.VMEM((RW * C,), f32),
            pltpu.VMEM((RW * C,), f32),
        ],
    )(cos.reshape(B * C))
    knn = mask_flat.reshape(B, C)

    ens_o = pl.pallas_call(
        _tc2_kernel,
        grid=grid,
        in_specs=[
            pl.BlockSpec((TB, D), lambda i: (i, 0)),
            pl.BlockSpec((TB, C), lambda i: (i, 0)),
            pl.BlockSpec((TB, C), lambda i: (i, 0)),
            pl.BlockSpec((O * C, D), lambda i: (0, 0)),
            pl.BlockSpec((1, O * C), lambda i: (0, 0)),
            pl.BlockSpec((O * C, O), lambda i: (0, 0)),
        ],
        out_specs=pl.BlockSpec((TB, O), lambda i: (i, 0)),
        out_shape=jax.ShapeDtypeStruct((B, O), f32),
    )(x, cos, knn, wm_perm, bm_perm, sel)

    return (ens_o, tanh_o, van, cosd, knn)


# speedup vs baseline: 1.9073x; 1.0791x over previous
import jax
import jax.numpy as jnp
from jax import lax
from jax.experimental import pallas as pl
from jax.experimental.pallas import tpu as pltpu
from jax.experimental.pallas import tpu_sc as plsc

B = 16384
D = 128
C = 64
O = 10
K = 8
TB = 2048
NC = 2
NS = 16
NW = NC * NS
RW = B // NW


def _tc1_kernel(x_ref, keys_ref, wv_ref, bv_ref, wt_ref, bt_ref,
                cos_out, cosd_out, van_out, tanh_out):
    x = x_ref[...]
    norm = jnp.sqrt(jnp.sum(x * x, axis=1, keepdims=True))
    xn = x / jnp.maximum(norm, 1e-12)

    dn = (((1,), (1,)), ((), ()))
    cos = lax.dot_general(xn, keys_ref[...], dn,
                          preferred_element_type=jnp.float32)
    cos_out[...] = cos
    cosd_out[...] = 1.0 - cos

    lv = lax.dot_general(x, wv_ref[...], dn,
                         preferred_element_type=jnp.float32) + bv_ref[...]
    m = jnp.max(lv, axis=1, keepdims=True)
    sh = lv - m
    van_out[...] = sh - jnp.log(jnp.sum(jnp.exp(sh), axis=1, keepdims=True))

    lt = lax.dot_general(x, wt_ref[...], dn,
                         preferred_element_type=jnp.float32) + bt_ref[...]
    tanh_out[...] = jnp.tanh(lt)


_PERM_DNUMS = lax.GatherDimensionNumbers(
    offset_dims=(), collapsed_slice_dims=(0,), start_index_map=(0,))


def _lane_perm(v, perm_idx):
    return lax.gather(v, perm_idx[:, None], dimension_numbers=_PERM_DNUMS,
                      slice_sizes=(1,),
                      mode=lax.GatherScatterMode.PROMISE_IN_BOUNDS)


def _sc_topk_body(cos_hbm, mask_hbm, in_v, out_v):
    wid = lax.axis_index("s") * NC + lax.axis_index("c")
    base = wid * (RW * C)
    pltpu.sync_copy(cos_hbm.at[pl.ds(base, RW * C)], in_v)

    iota = lax.broadcasted_iota(jnp.int32, (16,), 0)
    perms = [iota ^ p for p in (8, 4, 2, 1)]
    lane_ids = [iota + 16 * j for j in range(4)]
    neg_inf = jnp.full((16,), -jnp.inf, jnp.float32)
    big = jnp.full((16,), 1000, jnp.int32)
    ones = jnp.full((16,), 1.0, jnp.float32)
    zeros = jnp.zeros((16,), jnp.float32)

    @plsc.parallel_loop(0, RW, step=1, unroll=8)
    def row_body(r):
        off = r * C
        vs = [in_v[pl.ds(off + 16 * j, 16)] for j in range(4)]
        ms = [zeros, zeros, zeros, zeros]
        for _ in range(K):
            m = jnp.maximum(jnp.maximum(vs[0], vs[1]),
                            jnp.maximum(vs[2], vs[3]))
            for p in perms:
                m = jnp.maximum(m, _lane_perm(m, p))
            cands = [jnp.where(vs[j] == m, lane_ids[j], big)
                     for j in range(4)]
            g = jnp.minimum(jnp.minimum(cands[0], cands[1]),
                            jnp.minimum(cands[2], cands[3]))
            for p in perms:
                g = jnp.minimum(g, _lane_perm(g, p))
            for j in range(4):
                s = lane_ids[j] == g
                vs[j] = jnp.where(s, neg_inf, vs[j])
                ms[j] = jnp.where(s, ones, ms[j])
        for j in range(4):
            out_v[pl.ds(off + 16 * j, 16)] = ms[j]

    pltpu.sync_copy(out_v, mask_hbm.at[pl.ds(base, RW * C)])


def _tc2_kernel(x_ref, cos_ref, mask_ref, wm_ref, bm_ref, sel_ref, ens_out):
    x = x_ref[...]
    cos = cos_ref[...]
    maskf = mask_ref[...]
    w = cos * maskf
    denom = jnp.sum(w, axis=1, keepdims=True)

    dn = (((1,), (1,)), ((), ()))
    z = lax.dot_general(x, wm_ref[...], dn,
                        preferred_element_type=jnp.float32) + bm_ref[...]
    ens = jnp.tanh(z)
    w_tiled = jnp.concatenate([w] * O, axis=1)
    p = ens * w_tiled
    num = lax.dot_general(
        p, sel_ref[...], (((1,), (0,)), ((), ())),
        preferred_element_type=jnp.float32)
    ens_out[...] = num / denom


@jax.jit
def kernel(x, keys, W_models, b_models, W_van, b_van, W_tanh, b_tanh):
    wm_perm = W_models.transpose(1, 0, 2).reshape(O * C, D)
    bm_perm = b_models.T.reshape(1, O * C)
    sel = (jnp.arange(O * C)[:, None] // C ==
           jnp.arange(O)[None, :]).astype(jnp.float32)
    bv = b_van.reshape(1, O)
    bt = b_tanh.reshape(1, O)

    grid = (B // TB,)
    f32 = jnp.float32

    cos, cosd, van, tanh_o = pl.pallas_call(
        _tc1_kernel,
        grid=grid,
        in_specs=[
            pl.BlockSpec((TB, D), lambda i: (i, 0)),
            pl.BlockSpec((C, D), lambda i: (0, 0)),
            pl.BlockSpec((O, D), lambda i: (0, 0)),
            pl.BlockSpec((1, O), lambda i: (0, 0)),
            pl.BlockSpec((O, D), lambda i: (0, 0)),
            pl.BlockSpec((1, O), lambda i: (0, 0)),
        ],
        out_specs=[
            pl.BlockSpec((TB, C), lambda i: (i, 0)),
            pl.BlockSpec((TB, C), lambda i: (i, 0)),
            pl.BlockSpec((TB, O), lambda i: (i, 0)),
            pl.BlockSpec((TB, O), lambda i: (i, 0)),
        ],
        out_shape=[
            jax.ShapeDtypeStruct((B, C), f32),
            jax.ShapeDtypeStruct((B, C), f32),
            jax.ShapeDtypeStruct((B, O), f32),
            jax.ShapeDtypeStruct((B, O), f32),
        ],
    )(x, keys, W_van, bv, W_tanh, bt)

    mesh = plsc.VectorSubcoreMesh(core_axis_name="c", subcore_axis_name="s",
                                  num_cores=NC, num_subcores=NS)
    mask_flat = pl.kernel(
        _sc_topk_body,
        out_type=jax.ShapeDtypeStruct((B * C,), f32),
        mesh=mesh,
        scratch_types=[
            pltpu.VMEM((RW * C,), f32),
            pltpu.VMEM((RW * C,), f32),
        ],
    )(cos.reshape(B * C))
    knn = mask_flat.reshape(B, C)

    ens_o = pl.pallas_call(
        _tc2_kernel,
        grid=grid,
        in_specs=[
            pl.BlockSpec((TB, D), lambda i: (i, 0)),
            pl.BlockSpec((TB, C), lambda i: (i, 0)),
            pl.BlockSpec((TB, C), lambda i: (i, 0)),
            pl.BlockSpec((O * C, D), lambda i: (0, 0)),
            pl.BlockSpec((1, O * C), lambda i: (0, 0)),
            pl.BlockSpec((O * C, O), lambda i: (0, 0)),
        ],
        out_specs=pl.BlockSpec((TB, O), lambda i: (i, 0)),
        out_shape=jax.ShapeDtypeStruct((B, O), f32),
    )(x, cos, knn, wm_perm, bm_perm, sel)

    return (ens_o, tanh_o, van, cosd, knn)


# speedup vs baseline: 1.9453x; 1.0199x over previous
import jax
import jax.numpy as jnp
from jax import lax
from jax.experimental import pallas as pl
from jax.experimental.pallas import tpu as pltpu
from jax.experimental.pallas import tpu_sc as plsc

B = 16384
D = 128
C = 64
O = 10
K = 8
TB = 4096
NC = 2
NS = 16
NW = NC * NS
RW = B // NW


def _tc1_kernel(x_ref, keys_ref, wv_ref, bv_ref, wt_ref, bt_ref,
                cos_out, cosd_out, van_out, tanh_out):
    x = x_ref[...]
    norm = jnp.sqrt(jnp.sum(x * x, axis=1, keepdims=True))
    xn = x / jnp.maximum(norm, 1e-12)

    dn = (((1,), (1,)), ((), ()))
    cos = lax.dot_general(xn, keys_ref[...], dn,
                          preferred_element_type=jnp.float32)
    cos_out[...] = cos
    cosd_out[...] = 1.0 - cos

    lv = lax.dot_general(x, wv_ref[...], dn,
                         preferred_element_type=jnp.float32) + bv_ref[...]
    m = jnp.max(lv, axis=1, keepdims=True)
    sh = lv - m
    van_out[...] = sh - jnp.log(jnp.sum(jnp.exp(sh), axis=1, keepdims=True))

    lt = lax.dot_general(x, wt_ref[...], dn,
                         preferred_element_type=jnp.float32) + bt_ref[...]
    tanh_out[...] = jnp.tanh(lt)


_PERM_DNUMS = lax.GatherDimensionNumbers(
    offset_dims=(), collapsed_slice_dims=(0,), start_index_map=(0,))


def _lane_perm(v, perm_idx):
    return lax.gather(v, perm_idx[:, None], dimension_numbers=_PERM_DNUMS,
                      slice_sizes=(1,),
                      mode=lax.GatherScatterMode.PROMISE_IN_BOUNDS)


def _sc_topk_body(cos_hbm, mask_hbm, in_v, out_v):
    wid = lax.axis_index("s") * NC + lax.axis_index("c")
    base = wid * (RW * C)
    pltpu.sync_copy(cos_hbm.at[pl.ds(base, RW * C)], in_v)

    iota = lax.broadcasted_iota(jnp.int32, (16,), 0)
    perms = [iota ^ p for p in (8, 4, 2, 1)]
    lane_ids = [iota + 16 * j for j in range(4)]
    neg_inf = jnp.full((16,), -jnp.inf, jnp.float32)
    big = jnp.full((16,), 1000, jnp.int32)
    ones = jnp.full((16,), 1.0, jnp.float32)
    zeros = jnp.zeros((16,), jnp.float32)

    @plsc.parallel_loop(0, RW, step=1, unroll=8)
    def row_body(r):
        off = r * C
        vs = [in_v[pl.ds(off + 16 * j, 16)] for j in range(4)]
        ms = [zeros, zeros, zeros, zeros]
        for _ in range(K):
            m = jnp.maximum(jnp.maximum(vs[0], vs[1]),
                            jnp.maximum(vs[2], vs[3]))
            for p in perms:
                m = jnp.maximum(m, _lane_perm(m, p))
            cands = [jnp.where(vs[j] == m, lane_ids[j], big)
                     for j in range(4)]
            g = jnp.minimum(jnp.minimum(cands[0], cands[1]),
                            jnp.minimum(cands[2], cands[3]))
            for p in perms:
                g = jnp.minimum(g, _lane_perm(g, p))
            for j in range(4):
                s = lane_ids[j] == g
                vs[j] = jnp.where(s, neg_inf, vs[j])
                ms[j] = jnp.where(s, ones, ms[j])
        for j in range(4):
            out_v[pl.ds(off + 16 * j, 16)] = ms[j]

    pltpu.sync_copy(out_v, mask_hbm.at[pl.ds(base, RW * C)])


def _tc2_kernel(x_ref, cos_ref, mask_ref, wm_ref, bm_ref, sel_ref, ens_out):
    x = x_ref[...]
    cos = cos_ref[...]
    maskf = mask_ref[...]
    w = cos * maskf
    denom = jnp.sum(w, axis=1, keepdims=True)

    dn = (((1,), (1,)), ((), ()))
    z = lax.dot_general(x, wm_ref[...], dn,
                        preferred_element_type=jnp.float32) + bm_ref[...]
    ens = jnp.tanh(z)
    w_tiled = jnp.concatenate([w] * O, axis=1)
    p = ens * w_tiled
    num = lax.dot_general(
        p, sel_ref[...], (((1,), (0,)), ((), ())),
        preferred_element_type=jnp.float32)
    ens_out[...] = num / denom


@jax.jit
def kernel(x, keys, W_models, b_models, W_van, b_van, W_tanh, b_tanh):
    wm_perm = W_models.transpose(1, 0, 2).reshape(O * C, D)
    bm_perm = b_models.T.reshape(1, O * C)
    sel = (jnp.arange(O * C)[:, None] // C ==
           jnp.arange(O)[None, :]).astype(jnp.float32)
    bv = b_van.reshape(1, O)
    bt = b_tanh.reshape(1, O)

    grid = (B // TB,)
    f32 = jnp.float32

    cos, cosd, van, tanh_o = pl.pallas_call(
        _tc1_kernel,
        grid=grid,
        in_specs=[
            pl.BlockSpec((TB, D), lambda i: (i, 0)),
            pl.BlockSpec((C, D), lambda i: (0, 0)),
            pl.BlockSpec((O, D), lambda i: (0, 0)),
            pl.BlockSpec((1, O), lambda i: (0, 0)),
            pl.BlockSpec((O, D), lambda i: (0, 0)),
            pl.BlockSpec((1, O), lambda i: (0, 0)),
        ],
        out_specs=[
            pl.BlockSpec((TB, C), lambda i: (i, 0)),
            pl.BlockSpec((TB, C), lambda i: (i, 0)),
            pl.BlockSpec((TB, O), lambda i: (i, 0)),
            pl.BlockSpec((TB, O), lambda i: (i, 0)),
        ],
        out_shape=[
            jax.ShapeDtypeStruct((B, C), f32),
            jax.ShapeDtypeStruct((B, C), f32),
            jax.ShapeDtypeStruct((B, O), f32),
            jax.ShapeDtypeStruct((B, O), f32),
        ],
    )(x, keys, W_van, bv, W_tanh, bt)

    mesh = plsc.VectorSubcoreMesh(core_axis_name="c", subcore_axis_name="s",
                                  num_cores=NC, num_subcores=NS)
    mask_flat = pl.kernel(
        _sc_topk_body,
        out_type=jax.ShapeDtypeStruct((B * C,), f32),
        mesh=mesh,
        scratch_types=[
            pltpu.VMEM((RW * C,), f32),
            pltpu.VMEM((RW * C,), f32),
        ],
    )(cos.reshape(B * C))
    knn = mask_flat.reshape(B, C)

    ens_o = pl.pallas_call(
        _tc2_kernel,
        grid=grid,
        in_specs=[
            pl.BlockSpec((TB, D), lambda i: (i, 0)),
            pl.BlockSpec((TB, C), lambda i: (i, 0)),
            pl.BlockSpec((TB, C), lambda i: (i, 0)),
            pl.BlockSpec((O * C, D), lambda i: (0, 0)),
            pl.BlockSpec((1, O * C), lambda i: (0, 0)),
            pl.BlockSpec((O * C, O), lambda i: (0, 0)),
        ],
        out_specs=pl.BlockSpec((TB, O), lambda i: (i, 0)),
        out_shape=jax.ShapeDtypeStruct((B, O), f32),
    )(x, cos, knn, wm_perm, bm_perm, sel)

    return (ens_o, tanh_o, van, cosd, knn)
